# Initial kernel scaffold; baseline (speedup 1.0000x reference)
import jax
import jax.numpy as jnp
from jax.experimental import pallas as pl


def kernel(x, edge_index, W_enc, b_enc, W1, b1, W2, b2):
    raise NotImplementedError("write your pallas kernel here")


# speedup vs baseline: 3.1075x; 3.1075x over previous
import functools

import jax
import jax.numpy as jnp
from jax import lax
from jax.experimental import pallas as pl
from jax.experimental.pallas import tpu as pltpu
from jax.experimental.pallas import tpu_sc as plsc

N = 10000
E = 320000
D = 128
H = 64

NC = 2
NS = 16
NW = NC * NS
L = 16

NPAD = 10240
RPS = NPAD // NS

EPW = E // NW
KB = 80
NB = EPW // KB

_mesh = plsc.VectorSubcoreMesh(core_axis_name="c", subcore_axis_name="s")


@functools.partial(
    pl.kernel,
    out_type=(jax.ShapeDtypeStruct((NC, NPAD, D), jnp.float32),
              jax.ShapeDtypeStruct((NW * NPAD,), jnp.float32)),
    mesh=_mesh,
    scratch_types=[
        pltpu.VMEM((KB,), jnp.int32),
        pltpu.VMEM((KB,), jnp.int32),
        pltpu.VMEM((KB, D), jnp.float32),
        pltpu.VMEM((NPAD,), jnp.float32),
        pltpu.VMEM_SHARED((NPAD, D), jnp.float32),
        pltpu.SemaphoreType.DMA,
    ],
    compiler_params=pltpu.CompilerParams(needs_layout_passes=False),
)
def _seg_sum(x_hbm, src_hbm, dst_hbm, agg_hbm, deg_hbm,
             src_v, dst_v, rows_v, hist_v, agg_sh, sem):
    cid = lax.axis_index("c")
    sid = lax.axis_index("s")
    wid = sid * NC + cid

    zero = jnp.zeros((L,), jnp.float32)
    ones = jnp.ones((L,), jnp.float32)

    def _zrow(r, _):
        for j in range(D // L):
            rows_v[r, pl.ds(j * L, L)] = zero
        return ()

    lax.fori_loop(0, KB, _zrow, ())

    def _zhist(g, _):
        hist_v[pl.ds(g * L, L)] = zero
        return ()

    lax.fori_loop(0, NPAD // L, _zhist, ())

    for c in range(RPS // KB):
        pltpu.sync_copy(rows_v, agg_sh.at[pl.ds(sid * RPS + c * KB, KB), :])

    plsc.subcore_barrier()

    def _batch(b, _):
        base = wid * EPW + b * KB
        pltpu.sync_copy(src_hbm.at[pl.ds(base, KB)], src_v)
        pltpu.sync_copy(dst_hbm.at[pl.ds(base, KB)], dst_v)
        pltpu.async_copy(x_hbm.at[src_v], rows_v, sem).wait()
        pltpu.sync_copy(rows_v, agg_sh.at[dst_v], add=True)
        for g in range(KB // L):
            dv = dst_v[pl.ds(g * L, L)]
            plsc.addupdate_scatter(hist_v, [dv], ones,
                                   mask=jnp.ones((L,), jnp.bool_))
        return ()

    lax.fori_loop(0, NB, _batch, ())
    plsc.subcore_barrier()

    pltpu.sync_copy(agg_sh.at[pl.ds(sid * RPS, RPS), :],
                    agg_hbm.at[cid, pl.ds(sid * RPS, RPS), :])
    pltpu.sync_copy(hist_v, deg_hbm.at[pl.ds(wid * NPAD, NPAD)])


def _node_body(agg_ref, deg_ref, wenc_ref, benc_ref, w1a_ref, w1b_ref, b1_ref,
               c_ref):
    deg = jnp.maximum(jnp.sum(deg_ref[...], axis=0), 1.0)
    s = agg_ref[0] + agg_ref[1]
    xb = s / deg
    ne = jnp.dot(xb, wenc_ref[...], preferred_element_type=jnp.float32)
    ne = jnp.maximum(ne + benc_ref[...][None, :], 0.0)
    a = jnp.dot(ne, w1a_ref[...], preferred_element_type=jnp.float32)
    b = (jnp.dot(ne, w1b_ref[...], preferred_element_type=jnp.float32)
         + b1_ref[...][None, :])
    c_ref[...] = jnp.concatenate([a, b], axis=1)


_RB = 1024


def _node_stage(agg2, deg2, W_enc, b_enc, W1a, W1b, b1):
    grid = NPAD // _RB
    return pl.pallas_call(
        _node_body,
        grid=(grid,),
        in_specs=[
            pl.BlockSpec((NC, _RB, D), lambda i: (0, i, 0)),
            pl.BlockSpec((NW, _RB, 1), lambda i: (0, i, 0)),
            pl.BlockSpec((D, D), lambda i: (0, 0)),
            pl.BlockSpec((D,), lambda i: (0,)),
            pl.BlockSpec((D, H), lambda i: (0, 0)),
            pl.BlockSpec((D, H), lambda i: (0, 0)),
            pl.BlockSpec((H,), lambda i: (0,)),
        ],
        out_specs=pl.BlockSpec((_RB, D), lambda i: (i, 0)),
        out_shape=jax.ShapeDtypeStruct((NPAD, D), jnp.float32),
    )(agg2, deg2, W_enc, b_enc, W1a, W1b, b1)


@functools.partial(
    pl.kernel,
    out_type=jax.ShapeDtypeStruct((E,), jnp.float32),
    mesh=_mesh,
    scratch_types=[
        pltpu.VMEM((KB,), jnp.int32),
        pltpu.VMEM((KB,), jnp.int32),
        pltpu.VMEM((KB, D), jnp.float32),
        pltpu.VMEM((KB, D), jnp.float32),
        pltpu.VMEM((KB,), jnp.float32),
        pltpu.VMEM((H,), jnp.float32),
        pltpu.VMEM((L,), jnp.float32),
        pltpu.SemaphoreType.DMA,
    ],
    compiler_params=pltpu.CompilerParams(needs_layout_passes=False),
)
def _edge_mlp(c_hbm, src_hbm, dst_hbm, w2_hbm, bvec_hbm, out_hbm,
              src_v, dst_v, cs_v, cd_v, out_v, w2_v, bvec_v, sem):
    cid = lax.axis_index("c")
    sid = lax.axis_index("s")
    wid = sid * NC + cid

    pltpu.sync_copy(w2_hbm, w2_v)
    pltpu.sync_copy(bvec_hbm, bvec_v)
    w2 = [w2_v[pl.ds(j * L, L)] for j in range(H // L)]
    bvec = bvec_v[...]
    lane0 = jnp.arange(L, dtype=jnp.int32) == 0

    def _edge(i, _):
        acc = bvec
        for j in range(H // L):
            a = cs_v[i, pl.ds(j * L, L)]
            b = cd_v[i, pl.ds(H + j * L, L)]
            acc = acc + jnp.maximum(a + b, 0.0) * w2[j]
        r = jnp.sum(acc)
        plsc.store_scatter(out_v, [jnp.full((L,), i, jnp.int32)],
                           jnp.full((L,), r, jnp.float32), mask=lane0)
        return ()

    def _batch(b, _):
        base = wid * EPW + b * KB
        pltpu.sync_copy(src_hbm.at[pl.ds(base, KB)], src_v)
        pltpu.sync_copy(dst_hbm.at[pl.ds(base, KB)], dst_v)
        pltpu.async_copy(c_hbm.at[src_v], cs_v, sem).wait()
        pltpu.async_copy(c_hbm.at[dst_v], cd_v, sem).wait()
        lax.fori_loop(0, KB, _edge, ())
        pltpu.sync_copy(out_v, out_hbm.at[pl.ds(base, KB)])
        return ()

    lax.fori_loop(0, NB, _batch, ())


def kernel(x, edge_index, W_enc, b_enc, W1, b1, W2, b2):
    src = edge_index[0]
    dst = edge_index[1]
    agg2, deg_flat = _seg_sum(x, src, dst)
    deg2 = deg_flat.reshape(NW, NPAD, 1)
    C = _node_stage(agg2, deg2, W_enc, b_enc, W1[:D], W1[D:], b1)
    bvec = jnp.zeros((L,), jnp.float32).at[0].set(b2[0])
    logits = _edge_mlp(C, src, dst, W2.reshape(H), bvec)
    return logits.reshape(E, 1)
